# Initial kernel scaffold; baseline (speedup 1.0000x reference)
import jax
import jax.numpy as jnp
from jax.experimental import pallas as pl


def kernel(center, outside, negative, emb_center, emb_outside):
    raise NotImplementedError("write your pallas kernel here")


# speedup vs baseline: 1.0760x; 1.0760x over previous
import functools

import jax
import jax.numpy as jnp
from jax import lax
from jax.experimental import pallas as pl
from jax.experimental.pallas import tpu as pltpu
from jax.experimental.pallas import tpu_sc as plsc

V = 1000000
D = 32
B = 16384
K = 20

NC = 2
NS = 16
NW = NC * NS
NB = B // NW
C = 128
NCHUNK = NB // C


def _sc_body(emb_c, emb_o, cidx_h, oidx_h, nidx_h, uo_h, ng_h,
             cidx_v, oidx_v, nidx_v, crows, orows, nrows, uo_v, ng_v, sem):
    wid = lax.axis_index("s") * NC + lax.axis_index("c")
    pltpu.sync_copy(cidx_h.at[pl.ds(wid * NB, NB)], cidx_v)
    pltpu.sync_copy(oidx_h.at[pl.ds(wid * NB, NB)], oidx_v)
    pltpu.sync_copy(nidx_h.at[pl.ds(wid * NB * K, NB * K)], nidx_v)

    for ch in range(NCHUNK):
        cps = [
            pltpu.async_copy(emb_c.at[cidx_v.at[pl.ds(ch * C, C)]], crows, sem),
            pltpu.async_copy(emb_o.at[oidx_v.at[pl.ds(ch * C, C)]], orows, sem),
        ]
        for j in range(K):
            cps.append(pltpu.async_copy(
                emb_o.at[nidx_v.at[pl.ds((ch * K + j) * C, C)]],
                nrows.at[pl.ds(j * C, C)], sem))
        for cp in cps:
            cp.wait()

        def bbody(b, _):
            c0 = crows[b, pl.ds(0, 16)]
            c1 = crows[b, pl.ds(16, 16)]
            o0 = orows[b, pl.ds(0, 16)]
            o1 = orows[b, pl.ds(16, 16)]
            uo_v[b, :] = c0 * o0 + c1 * o1
            base = b * K
            a0 = nrows[base, pl.ds(0, 16)]
            a1 = nrows[base, pl.ds(16, 16)]
            for k in range(1, K):
                a0 = a0 + nrows[base + k, pl.ds(0, 16)]
                a1 = a1 + nrows[base + k, pl.ds(16, 16)]
            ng_v[b, :] = c0 * a0 + c1 * a1
            return 0

        lax.fori_loop(0, C, bbody, 0)
        out_base = wid * NB + ch * C
        pltpu.sync_copy(uo_v, uo_h.at[pl.ds(out_base, C)])
        pltpu.sync_copy(ng_v, ng_h.at[pl.ds(out_base, C)])


@jax.jit
def _sc_partials(cidx, oidx, nidx, emb_center, emb_outside):
    mesh = plsc.VectorSubcoreMesh(core_axis_name="c", subcore_axis_name="s")
    f32 = jnp.float32
    return pl.kernel(
        _sc_body,
        out_type=(
            jax.ShapeDtypeStruct((B, 16), f32),
            jax.ShapeDtypeStruct((B, 16), f32),
        ),
        mesh=mesh,
        compiler_params=pltpu.CompilerParams(use_tc_tiling_on_sc=False),
        scratch_types=[
            pltpu.VMEM((NB,), jnp.int32),
            pltpu.VMEM((NB,), jnp.int32),
            pltpu.VMEM((NB * K,), jnp.int32),
            pltpu.VMEM((C, D), f32),
            pltpu.VMEM((C, D), f32),
            pltpu.VMEM((K * C, D), f32),
            pltpu.VMEM((C, 16), f32),
            pltpu.VMEM((C, 16), f32),
            pltpu.SemaphoreType.DMA,
        ],
    )(emb_center, emb_outside, cidx, oidx, nidx)


def _tc_body(uo_ref, ng_ref, out_ref):
    uo = uo_ref[...]
    ng = ng_ref[...]
    gi = lax.broadcasted_iota(jnp.int32, (128, 8), 0) // 16
    gj = lax.broadcasted_iota(jnp.int32, (128, 8), 1)
    g = (gi == gj).astype(jnp.float32)
    dn = (((1,), (0,)), ((), ()))
    uos = lax.dot_general(uo, g, dn, preferred_element_type=jnp.float32)
    ngs = lax.dot_general(ng, g, dn, preferred_element_type=jnp.float32)

    def logsig(t):
        return jnp.minimum(t, 0.0) - jnp.log1p(jnp.exp(-jnp.abs(t)))

    loss = logsig(uos) + logsig(-ngs)
    out_ref[0, 0] = -jnp.sum(loss) / jnp.float32(B)


@jax.jit
def _tc_loss(uo2d, ng2d):
    return pl.pallas_call(
        _tc_body,
        out_shape=jax.ShapeDtypeStruct((1, 1), jnp.float32),
        out_specs=pl.BlockSpec(memory_space=pltpu.SMEM),
    )(uo2d, ng2d)


def kernel(center, outside, negative, emb_center, emb_outside):
    cidx = center.reshape(B)
    oidx = outside.reshape(B)
    nidx = negative.reshape(B * K)
    uo, ng = _sc_partials(cidx, oidx, nidx, emb_center, emb_outside)
    out = _tc_loss(uo.reshape(B * 16 // 128, 128), ng.reshape(B * 16 // 128, 128))
    return out[0, 0]
